# baseline (speedup 1.0000x reference)
import functools

import jax
import jax.numpy as jnp
from jax import lax
from jax.experimental import pallas as pl
from jax.experimental.pallas import tpu as pltpu
from jax.experimental.pallas import tpu_sc as plsc

_N_NODES = 10000
_N_EDGES = 20000
_N_INC = 320000
_D = 128
_DH = 64
_DQ = 32
_ALPHA = 0.5
_NTILES = 16
_CH = 128
_PPT = 20480
_NCH = _PPT // _CH
_NB = 32
_NBLK = _NCH // _NB
_INC_PAD = _NTILES * _PPT
_NODE_PAD = 10240
_XE_PAD = 20480
_XE_PT = _XE_PAD // _NTILES
_S_PAD = 10240
_S_PT = _S_PAD // _NTILES
_DEGW = 8

_MESH = plsc.VectorSubcoreMesh(core_axis_name="c", subcore_axis_name="s")
_SC_PARAMS = pltpu.CompilerParams(use_tc_tiling_on_sc=False)


def _mm1_body(x_ref, w_ref, b_ref, o0_ref, o1_ref, o2_ref, o3_ref):
    y = jnp.dot(x_ref[...], w_ref[...], preferred_element_type=jnp.float32)
    y = y + b_ref[...]
    o0_ref[...] = y[:, 0 * _DQ:1 * _DQ]
    o1_ref[...] = y[:, 1 * _DQ:2 * _DQ]
    o2_ref[...] = y[:, 2 * _DQ:3 * _DQ]
    o3_ref[...] = y[:, 3 * _DQ:4 * _DQ]


def _xw1(X, W1_w, W1_b):
    blk = 1024
    return pl.pallas_call(
        _mm1_body,
        grid=(_NODE_PAD // blk,),
        in_specs=[
            pl.BlockSpec((blk, _D), lambda i: (i, 0)),
            pl.BlockSpec((_D, _D), lambda i: (0, 0)),
            pl.BlockSpec((1, _D), lambda i: (0, 0)),
        ],
        out_specs=[pl.BlockSpec((blk, _DQ), lambda i: (i, 0))] * 4,
        out_shape=[jax.ShapeDtypeStruct((_NODE_PAD, _DQ), jnp.float32)] * 4,
    )(X, W1_w, W1_b)


def _sc_body(xw1q0, xw1q1, xw1q2, xw1q3, vtx3, edg3, zrow, ones_h,
             s_out, deg_out,
             vtx_v, edg_v, hbuf0, hbuf1, ones_v,
             xe_acc0, xe_acc1, deg_acc, s_acc, gsem, gsem2):
    c = lax.axis_index("c")
    s = lax.axis_index("s")

    pltpu.sync_copy(zrow, hbuf0)
    for k in range(_XE_PT // _CH):
        rows = pl.ds(s * _XE_PT + k * _CH, _CH)
        pltpu.sync_copy(hbuf0, xe_acc0.at[rows])
        pltpu.sync_copy(hbuf0, xe_acc1.at[rows])

    @pl.when(c == 0)
    def _():
        for k in range(_S_PT // _CH):
            pltpu.sync_copy(hbuf0.at[:, pl.ds(0, _DEGW)],
                            deg_acc.at[pl.ds(s * _S_PT + k * _CH, _CH)])
        pltpu.sync_copy(ones_h, ones_v)

    plsc.subcore_barrier()

    def runA(xw1h, xe_acc, with_deg):
        def block(b, carry):
            pltpu.sync_copy(vtx3.at[s, pl.ds(b * _NB, _NB)], vtx_v)
            pltpu.sync_copy(edg3.at[s, pl.ds(b * _NB, _NB)], edg_v)

            def pair(p, c2):
                j0 = 2 * p
                j1 = j0 + 1
                cp0 = pltpu.async_copy(xw1h.at[vtx_v.at[j0]], hbuf0, gsem)
                cp1 = pltpu.async_copy(xw1h.at[vtx_v.at[j1]], hbuf1, gsem2)
                cp0.wait()
                pltpu.sync_copy(hbuf0, xe_acc.at[edg_v.at[j0]], add=True)
                if with_deg:
                    pltpu.sync_copy(ones_v, deg_acc.at[vtx_v.at[j0]], add=True)
                cp1.wait()
                pltpu.sync_copy(hbuf1, xe_acc.at[edg_v.at[j1]], add=True)
                if with_deg:
                    pltpu.sync_copy(ones_v, deg_acc.at[vtx_v.at[j1]], add=True)
                return c2
            lax.fori_loop(0, _NB // 2, pair, 0)
            return carry
        lax.fori_loop(0, _NBLK, block, 0)

    @pl.when(c == 0)
    def _():
        runA(xw1q0, xe_acc0, True)
        runA(xw1q1, xe_acc1, False)

    @pl.when(c == 1)
    def _():
        runA(xw1q2, xe_acc0, False)
        runA(xw1q3, xe_acc1, False)

    plsc.subcore_barrier()

    @pl.when(c == 0)
    def _():
        for k in range(_S_PT // _CH):
            rows = pl.ds(s * _S_PT + k * _CH, _CH)
            pltpu.sync_copy(deg_acc.at[rows], hbuf0.at[:, pl.ds(0, _DEGW)])
            pltpu.sync_copy(hbuf0.at[:, pl.ds(0, _DEGW)], deg_out.at[rows])

    for q, xe_acc in ((0, xe_acc0), (1, xe_acc1)):
        pltpu.sync_copy(zrow, hbuf0)
        for k in range(_S_PT // _CH):
            pltpu.sync_copy(hbuf0, s_acc.at[pl.ds(s * _S_PT + k * _CH, _CH)])
        plsc.subcore_barrier()

        def blockS(b, carry):
            pltpu.sync_copy(vtx3.at[s, pl.ds(b * _NB, _NB)], vtx_v)
            pltpu.sync_copy(edg3.at[s, pl.ds(b * _NB, _NB)], edg_v)

            def pair(p, c2):
                j0 = 2 * p
                j1 = j0 + 1
                cp0 = pltpu.async_copy(xe_acc.at[edg_v.at[j0]], hbuf0, gsem)
                cp1 = pltpu.async_copy(xe_acc.at[edg_v.at[j1]], hbuf1, gsem2)
                cp0.wait()
                pltpu.sync_copy(hbuf0, s_acc.at[vtx_v.at[j0]], add=True)
                cp1.wait()
                pltpu.sync_copy(hbuf1, s_acc.at[vtx_v.at[j1]], add=True)
                return c2
            lax.fori_loop(0, _NB // 2, pair, 0)
            return carry
        lax.fori_loop(0, _NBLK, blockS, 0)
        plsc.subcore_barrier()

        for k in range(_S_PT // _CH):
            rows = pl.ds(s * _S_PT + k * _CH, _CH)
            pltpu.sync_copy(s_acc.at[rows], hbuf0)
            pltpu.sync_copy(hbuf0, s_out.at[rows, pl.ds(c * _DH + q * _DQ, _DQ)])
        plsc.subcore_barrier()


_sc_fused = functools.partial(
    pl.kernel,
    out_type=[
        jax.ShapeDtypeStruct((_S_PAD, _D), jnp.float32),
        jax.ShapeDtypeStruct((_S_PAD, _DEGW), jnp.float32),
    ],
    mesh=_MESH,
    compiler_params=_SC_PARAMS,
    scratch_types=[
        pltpu.VMEM((_NB, _CH), jnp.int32),
        pltpu.VMEM((_NB, _CH), jnp.int32),
        pltpu.VMEM((_CH, _DQ), jnp.float32),
        pltpu.VMEM((_CH, _DQ), jnp.float32),
        pltpu.VMEM((_CH, _DEGW), jnp.float32),
        pltpu.VMEM_SHARED((_XE_PAD, _DQ), jnp.float32),
        pltpu.VMEM_SHARED((_XE_PAD, _DQ), jnp.float32),
        pltpu.VMEM_SHARED((_S_PAD, _DEGW), jnp.float32),
        pltpu.VMEM_SHARED((_S_PAD, _DQ), jnp.float32),
        pltpu.SemaphoreType.DMA,
        pltpu.SemaphoreType.DMA,
    ],
)(_sc_body)


def _final_body(x_ref, x0_ref, s_ref, deg_ref,
                w2a_ref, w2b_ref, b2_ref, ww_ref, wb_ref, o_ref):
    d = deg_ref[...][:, 0:1]
    xv = jnp.dot(x_ref[...] * d, w2a_ref[...], preferred_element_type=jnp.float32)
    xv = xv + jnp.dot(s_ref[...], w2b_ref[...], preferred_element_type=jnp.float32)
    xv = xv + d * b2_ref[...]
    xmix = (1.0 - _ALPHA) * xv + _ALPHA * x0_ref[...]
    o_ref[...] = jnp.dot(xmix, ww_ref[...], preferred_element_type=jnp.float32) + wb_ref[...]


def _final(X, X0, S, deg, W2a, W2b, b2, W_w, W_b):
    blk = 1000
    full = lambda i: (0, 0)
    return pl.pallas_call(
        _final_body,
        grid=(_N_NODES // blk,),
        in_specs=[
            pl.BlockSpec((blk, _D), lambda i: (i, 0)),
            pl.BlockSpec((blk, _D), lambda i: (i, 0)),
            pl.BlockSpec((blk, _D), lambda i: (i, 0)),
            pl.BlockSpec((blk, _DEGW), lambda i: (i, 0)),
            pl.BlockSpec((_D, _D), full),
            pl.BlockSpec((_D, _D), full),
            pl.BlockSpec((1, _D), full),
            pl.BlockSpec((_D, _D), full),
            pl.BlockSpec((1, _D), full),
        ],
        out_specs=pl.BlockSpec((blk, _D), lambda i: (i, 0)),
        out_shape=jax.ShapeDtypeStruct((_N_NODES, _D), jnp.float32),
    )(X, X0, S, deg, W2a, W2b, b2, W_w, W_b)


def kernel(X, vertex, edges, X0, W1_w, W1_b, W2_w, W2_b, W_w, W_b):
    vertex = vertex.astype(jnp.int32)
    edges = edges.astype(jnp.int32)
    npad = _INC_PAD - _N_INC
    vtx3 = jnp.concatenate(
        [vertex, jnp.full((npad,), _N_NODES, jnp.int32)]).reshape(_NTILES, _NCH, _CH)
    edg3 = jnp.concatenate(
        [edges, jnp.full((npad,), _N_EDGES, jnp.int32)]).reshape(_NTILES, _NCH, _CH)

    Xp = jnp.pad(X, ((0, _NODE_PAD - _N_NODES), (0, 0)))
    xw1q0, xw1q1, xw1q2, xw1q3 = _xw1(Xp, W1_w, W1_b.reshape(1, _D))

    zrow = jnp.zeros((_CH, _DQ), jnp.float32)
    ones_h = jnp.ones((_CH, _DEGW), jnp.float32)

    S, deg = _sc_fused(xw1q0, xw1q1, xw1q2, xw1q3, vtx3, edg3, zrow, ones_h)
    S = S[:_N_NODES]
    deg = deg[:_N_NODES]

    out = _final(X, X0, S, deg,
                 W2_w[:_D], W2_w[_D:],
                 W2_b.reshape(1, _D), W_w, W_b.reshape(1, _D))
    return out


# speedup vs baseline: 1.3219x; 1.3219x over previous
import functools

import jax
import jax.numpy as jnp
from jax import lax
from jax.experimental import pallas as pl
from jax.experimental.pallas import tpu as pltpu
from jax.experimental.pallas import tpu_sc as plsc

_N_NODES = 10000
_N_EDGES = 20000
_N_INC = 320000
_D = 128
_DH = 64
_DQ = 32
_ALPHA = 0.5
_NTILES = 16
_PPT = _N_INC // _NTILES
_CH = 80
_NCH = _PPT // _CH
_NB = 50
_NBLK = _NCH // _NB
_ZCH = 160
_XE_PAD = 20480
_XE_PT = _XE_PAD // _NTILES
_S_PAD = 10240
_S_PT = _S_PAD // _NTILES
_DEGW = 8

_MESH = plsc.VectorSubcoreMesh(core_axis_name="c", subcore_axis_name="s")
_SC_PARAMS = pltpu.CompilerParams(use_tc_tiling_on_sc=False)


def _mm1_body(x_ref, w_ref, b_ref, o0_ref, o1_ref):
    y = jnp.dot(x_ref[...], w_ref[...], preferred_element_type=jnp.float32)
    y = y + b_ref[...]
    o0_ref[...] = y[:, :_DH]
    o1_ref[...] = y[:, _DH:]


def _xw1(X, W1_w, W1_b):
    blk = 1000
    return pl.pallas_call(
        _mm1_body,
        grid=(_N_NODES // blk,),
        in_specs=[
            pl.BlockSpec((blk, _D), lambda i: (i, 0)),
            pl.BlockSpec((_D, _D), lambda i: (0, 0)),
            pl.BlockSpec((1, _D), lambda i: (0, 0)),
        ],
        out_specs=[
            pl.BlockSpec((blk, _DH), lambda i: (i, 0)),
            pl.BlockSpec((blk, _DH), lambda i: (i, 0)),
        ],
        out_shape=[
            jax.ShapeDtypeStruct((_N_NODES, _DH), jnp.float32),
            jax.ShapeDtypeStruct((_N_NODES, _DH), jnp.float32),
        ],
    )(X, W1_w, W1_b)


def _sc1_body(xw1a, xw1b, vtx3, edg3, zrow, zdeg, ones_h,
              xe00, xe01, xe10, xe11, deg,
              vtx_v, edg_v, rowbuf, rowbuf2, ones_v, zbuf, qbuf, zdbuf,
              xe_acc, deg_acc, gsem, gsem2):
    c = lax.axis_index("c")
    s = lax.axis_index("s")
    pltpu.sync_copy(zrow, zbuf)
    for k in range(_XE_PT // _ZCH):
        pltpu.sync_copy(zbuf, xe_acc.at[pl.ds(s * _XE_PT + k * _ZCH, _ZCH)])

    @pl.when(c == 0)
    def _():
        pltpu.sync_copy(zdeg, zdbuf)
        pltpu.sync_copy(zdbuf, deg_acc.at[pl.ds(s * _S_PT, _S_PT)])
        pltpu.sync_copy(ones_h, ones_v)

    plsc.subcore_barrier()

    def run(xw1h, with_deg):
        def block(b, carry):
            pltpu.sync_copy(vtx3.at[s, pl.ds(b * _NB, _NB)], vtx_v)
            pltpu.sync_copy(edg3.at[s, pl.ds(b * _NB, _NB)], edg_v)

            def pair(p, c2):
                j0 = 2 * p
                j1 = j0 + 1
                cp0 = pltpu.async_copy(xw1h.at[vtx_v.at[j0]], rowbuf, gsem)
                cp1 = pltpu.async_copy(xw1h.at[vtx_v.at[j1]], rowbuf2, gsem2)
                cp0.wait()
                pltpu.sync_copy(rowbuf, xe_acc.at[edg_v.at[j0]], add=True)
                if with_deg:
                    pltpu.sync_copy(ones_v, deg_acc.at[vtx_v.at[j0]], add=True)
                cp1.wait()
                pltpu.sync_copy(rowbuf2, xe_acc.at[edg_v.at[j1]], add=True)
                if with_deg:
                    pltpu.sync_copy(ones_v, deg_acc.at[vtx_v.at[j1]], add=True)
                return c2
            lax.fori_loop(0, _NB // 2, pair, 0)
            return carry
        lax.fori_loop(0, _NBLK, block, 0)

    @pl.when(c == 0)
    def _():
        run(xw1a, True)

    @pl.when(c == 1)
    def _():
        run(xw1b, False)

    plsc.subcore_barrier()

    def dump(xeq0, xeq1):
        for q, xeq in ((0, xeq0), (1, xeq1)):
            for k in range(_XE_PT // _ZCH):
                rows = pl.ds(s * _XE_PT + k * _ZCH, _ZCH)
                pltpu.sync_copy(xe_acc.at[rows, pl.ds(q * _DQ, _DQ)], qbuf)
                pltpu.sync_copy(qbuf, xeq.at[rows])

    @pl.when(c == 0)
    def _():
        dump(xe00, xe01)
        pltpu.sync_copy(deg_acc.at[pl.ds(s * _S_PT, _S_PT)], zdbuf)
        pltpu.sync_copy(zdbuf, deg.at[pl.ds(s * _S_PT, _S_PT)])

    @pl.when(c == 1)
    def _():
        dump(xe10, xe11)


_sc_phase1 = functools.partial(
    pl.kernel,
    out_type=[
        jax.ShapeDtypeStruct((_XE_PAD, _DQ), jnp.float32),
        jax.ShapeDtypeStruct((_XE_PAD, _DQ), jnp.float32),
        jax.ShapeDtypeStruct((_XE_PAD, _DQ), jnp.float32),
        jax.ShapeDtypeStruct((_XE_PAD, _DQ), jnp.float32),
        jax.ShapeDtypeStruct((_S_PAD, _DEGW), jnp.float32),
    ],
    mesh=_MESH,
    compiler_params=_SC_PARAMS,
    scratch_types=[
        pltpu.VMEM((_NB, _CH), jnp.int32),
        pltpu.VMEM((_NB, _CH), jnp.int32),
        pltpu.VMEM((_CH, _DH), jnp.float32),
        pltpu.VMEM((_CH, _DH), jnp.float32),
        pltpu.VMEM((_CH, _DEGW), jnp.float32),
        pltpu.VMEM((_ZCH, _DH), jnp.float32),
        pltpu.VMEM((_ZCH, _DQ), jnp.float32),
        pltpu.VMEM((_S_PT, _DEGW), jnp.float32),
        pltpu.VMEM_SHARED((_XE_PAD, _DH), jnp.float32),
        pltpu.VMEM_SHARED((_S_PAD, _DEGW), jnp.float32),
        pltpu.SemaphoreType.DMA,
        pltpu.SemaphoreType.DMA,
    ],
)(_sc1_body)


def _sc2_body(xe00, xe01, xe10, xe11, vtx3, edg3, zq,
              s_out,
              vtx_v, edg_v, rowbuf, rowbuf2, qbuf, dbuf, s_acc, gsem, gsem2):
    c = lax.axis_index("c")
    s = lax.axis_index("s")
    pltpu.sync_copy(zq, qbuf)

    def one_pass(xeq, col_off):
        for k in range(_S_PT // _ZCH):
            pltpu.sync_copy(qbuf, s_acc.at[pl.ds(s * _S_PT + k * _ZCH, _ZCH)])
        plsc.subcore_barrier()

        def block(b, carry):
            pltpu.sync_copy(vtx3.at[s, pl.ds(b * _NB, _NB)], vtx_v)
            pltpu.sync_copy(edg3.at[s, pl.ds(b * _NB, _NB)], edg_v)

            def pair(p, c2):
                j0 = 2 * p
                j1 = j0 + 1
                cp0 = pltpu.async_copy(xeq.at[edg_v.at[j0]], rowbuf, gsem)
                cp1 = pltpu.async_copy(xeq.at[edg_v.at[j1]], rowbuf2, gsem2)
                cp0.wait()
                pltpu.sync_copy(rowbuf, s_acc.at[vtx_v.at[j0]], add=True)
                cp1.wait()
                pltpu.sync_copy(rowbuf2, s_acc.at[vtx_v.at[j1]], add=True)
                return c2
            lax.fori_loop(0, _NB // 2, pair, 0)
            return carry
        lax.fori_loop(0, _NBLK, block, 0)
        plsc.subcore_barrier()
        for k in range(_S_PT // _ZCH):
            rows = pl.ds(s * _S_PT + k * _ZCH, _ZCH)
            pltpu.sync_copy(s_acc.at[rows], dbuf)
            pltpu.sync_copy(dbuf, s_out.at[rows, pl.ds(col_off, _DQ)])
        plsc.subcore_barrier()

    @pl.when(c == 0)
    def _():
        one_pass(xe00, 0)
        one_pass(xe01, _DQ)

    @pl.when(c == 1)
    def _():
        one_pass(xe10, _DH)
        one_pass(xe11, _DH + _DQ)


_sc_phase2 = functools.partial(
    pl.kernel,
    out_type=jax.ShapeDtypeStruct((_S_PAD, _D), jnp.float32),
    mesh=_MESH,
    compiler_params=_SC_PARAMS,
    scratch_types=[
        pltpu.VMEM((_NB, _CH), jnp.int32),
        pltpu.VMEM((_NB, _CH), jnp.int32),
        pltpu.VMEM((_CH, _DQ), jnp.float32),
        pltpu.VMEM((_CH, _DQ), jnp.float32),
        pltpu.VMEM((_ZCH, _DQ), jnp.float32),
        pltpu.VMEM((_ZCH, _DQ), jnp.float32),
        pltpu.VMEM_SHARED((_S_PAD, _DQ), jnp.float32),
        pltpu.SemaphoreType.DMA,
        pltpu.SemaphoreType.DMA,
    ],
)(_sc2_body)


def _final_body(x_ref, x0_ref, s_ref, deg_ref,
                w2a_ref, w2b_ref, b2_ref, ww_ref, wb_ref, o_ref):
    d = deg_ref[...][:, 0:1]
    xv = jnp.dot(x_ref[...] * d, w2a_ref[...], preferred_element_type=jnp.float32)
    xv = xv + jnp.dot(s_ref[...], w2b_ref[...], preferred_element_type=jnp.float32)
    xv = xv + d * b2_ref[...]
    xmix = (1.0 - _ALPHA) * xv + _ALPHA * x0_ref[...]
    o_ref[...] = jnp.dot(xmix, ww_ref[...], preferred_element_type=jnp.float32) + wb_ref[...]


def _final(X, X0, S, deg, W2a, W2b, b2, W_w, W_b):
    blk = 1000
    full = lambda i: (0, 0)
    return pl.pallas_call(
        _final_body,
        grid=(_N_NODES // blk,),
        in_specs=[
            pl.BlockSpec((blk, _D), lambda i: (i, 0)),
            pl.BlockSpec((blk, _D), lambda i: (i, 0)),
            pl.BlockSpec((blk, _D), lambda i: (i, 0)),
            pl.BlockSpec((blk, _DEGW), lambda i: (i, 0)),
            pl.BlockSpec((_D, _D), full),
            pl.BlockSpec((_D, _D), full),
            pl.BlockSpec((1, _D), full),
            pl.BlockSpec((_D, _D), full),
            pl.BlockSpec((1, _D), full),
        ],
        out_specs=pl.BlockSpec((blk, _D), lambda i: (i, 0)),
        out_shape=jax.ShapeDtypeStruct((_N_NODES, _D), jnp.float32),
    )(X, X0, S, deg, W2a, W2b, b2, W_w, W_b)


def kernel(X, vertex, edges, X0, W1_w, W1_b, W2_w, W2_b, W_w, W_b):
    vertex = vertex.astype(jnp.int32)
    edges = edges.astype(jnp.int32)
    vtx3 = vertex.reshape(_NTILES, _NCH, _CH)
    edg3 = edges.reshape(_NTILES, _NCH, _CH)

    xw1a, xw1b = _xw1(X, W1_w, W1_b.reshape(1, _D))

    zrow = jnp.zeros((_ZCH, _DH), jnp.float32)
    zq = jnp.zeros((_ZCH, _DQ), jnp.float32)
    zdeg = jnp.zeros((_S_PT, _DEGW), jnp.float32)
    ones_h = jnp.ones((_CH, _DEGW), jnp.float32)

    xe00, xe01, xe10, xe11, deg = _sc_phase1(
        xw1a, xw1b, vtx3, edg3, zrow, zdeg, ones_h)
    S = _sc_phase2(xe00, xe01, xe10, xe11, vtx3, edg3, zq)
    S = S[:_N_NODES]
    deg = deg[:_N_NODES]

    out = _final(X, X0, S, deg,
                 W2_w[:_D], W2_w[_D:],
                 W2_b.reshape(1, _D), W_w, W_b.reshape(1, _D))
    return out


# speedup vs baseline: 1.5774x; 1.1933x over previous
import functools

import jax
import jax.numpy as jnp
from jax import lax
from jax.experimental import pallas as pl
from jax.experimental.pallas import tpu as pltpu
from jax.experimental.pallas import tpu_sc as plsc

_N_NODES = 10000
_N_EDGES = 20000
_N_INC = 320000
_D = 128
_DH = 64
_ALPHA = 0.5
_NTILES = 16
_PPT = _N_INC // _NTILES
_CH = 80
_NCH = _PPT // _CH
_NB = 50
_NBLK = _NCH // _NB
_ZCH = 160
_XE_PAD = 20480
_XE_PT = _XE_PAD // _NTILES
_S_PAD = 10240
_S_PT = _S_PAD // _NTILES
_DEGW = 8

_MESH = plsc.VectorSubcoreMesh(core_axis_name="c", subcore_axis_name="s")
_SC_PARAMS = pltpu.CompilerParams(use_tc_tiling_on_sc=False)


def _mm1_body(x_ref, w_ref, b_ref, o0_ref, o1_ref):
    y = jnp.dot(x_ref[...], w_ref[...], preferred_element_type=jnp.float32)
    y = y + b_ref[...]
    o0_ref[...] = y[:, :_DH]
    o1_ref[...] = y[:, _DH:]


def _xw1(X, W1_w, W1_b):
    blk = 1000
    return pl.pallas_call(
        _mm1_body,
        grid=(_N_NODES // blk,),
        in_specs=[
            pl.BlockSpec((blk, _D), lambda i: (i, 0)),
            pl.BlockSpec((_D, _D), lambda i: (0, 0)),
            pl.BlockSpec((1, _D), lambda i: (0, 0)),
        ],
        out_specs=[
            pl.BlockSpec((blk, _DH), lambda i: (i, 0)),
            pl.BlockSpec((blk, _DH), lambda i: (i, 0)),
        ],
        out_shape=[
            jax.ShapeDtypeStruct((_N_NODES, _DH), jnp.float32),
            jax.ShapeDtypeStruct((_N_NODES, _DH), jnp.float32),
        ],
    )(X, W1_w, W1_b)


def _sc1_body(xw1a, xw1b, vtx3, edg3, zrow, zdeg, ones_h,
              xe0, xe1, deg,
              vtx_v, edg_v, rowbuf, rowbuf2, ones_v, zbuf, zdbuf,
              xe_acc, deg_acc, gsem, gsem2):
    c = lax.axis_index("c")
    s = lax.axis_index("s")
    pltpu.sync_copy(zrow, zbuf)
    for k in range(_XE_PT // _ZCH):
        pltpu.sync_copy(zbuf, xe_acc.at[pl.ds(s * _XE_PT + k * _ZCH, _ZCH)])

    @pl.when(c == 0)
    def _():
        pltpu.sync_copy(zdeg, zdbuf)
        pltpu.sync_copy(zdbuf, deg_acc.at[pl.ds(s * _S_PT, _S_PT)])
        pltpu.sync_copy(ones_h, ones_v)

    plsc.subcore_barrier()

    def run(xw1h, with_deg):
        def block(b, carry):
            pltpu.sync_copy(vtx3.at[s, pl.ds(b * _NB, _NB)], vtx_v)
            pltpu.sync_copy(edg3.at[s, pl.ds(b * _NB, _NB)], edg_v)

            def pair(p, c2):
                j0 = 2 * p
                j1 = j0 + 1
                cp0 = pltpu.async_copy(xw1h.at[vtx_v.at[j0]], rowbuf, gsem)
                cp1 = pltpu.async_copy(xw1h.at[vtx_v.at[j1]], rowbuf2, gsem2)
                cp0.wait()
                pltpu.sync_copy(rowbuf, xe_acc.at[edg_v.at[j0]], add=True)
                if with_deg:
                    pltpu.sync_copy(ones_v, deg_acc.at[vtx_v.at[j0]], add=True)
                cp1.wait()
                pltpu.sync_copy(rowbuf2, xe_acc.at[edg_v.at[j1]], add=True)
                if with_deg:
                    pltpu.sync_copy(ones_v, deg_acc.at[vtx_v.at[j1]], add=True)
                return c2
            lax.fori_loop(0, _NB // 2, pair, 0)
            return carry
        lax.fori_loop(0, _NBLK, block, 0)

    @pl.when(c == 0)
    def _():
        run(xw1a, True)

    @pl.when(c == 1)
    def _():
        run(xw1b, False)

    plsc.subcore_barrier()

    def dump(xe_h):
        for k in range(_XE_PT // _ZCH):
            rows = pl.ds(s * _XE_PT + k * _ZCH, _ZCH)
            pltpu.sync_copy(xe_acc.at[rows], zbuf)
            pltpu.sync_copy(zbuf, xe_h.at[rows])

    @pl.when(c == 0)
    def _():
        dump(xe0)
        pltpu.sync_copy(deg_acc.at[pl.ds(s * _S_PT, _S_PT)], zdbuf)
        pltpu.sync_copy(zdbuf, deg.at[pl.ds(s * _S_PT, _S_PT)])

    @pl.when(c == 1)
    def _():
        dump(xe1)


_sc_phase1 = functools.partial(
    pl.kernel,
    out_type=[
        jax.ShapeDtypeStruct((_XE_PAD, _DH), jnp.float32),
        jax.ShapeDtypeStruct((_XE_PAD, _DH), jnp.float32),
        jax.ShapeDtypeStruct((_S_PAD, _DEGW), jnp.float32),
    ],
    mesh=_MESH,
    compiler_params=_SC_PARAMS,
    scratch_types=[
        pltpu.VMEM((_NB, _CH), jnp.int32),
        pltpu.VMEM((_NB, _CH), jnp.int32),
        pltpu.VMEM((_CH, _DH), jnp.float32),
        pltpu.VMEM((_CH, _DH), jnp.float32),
        pltpu.VMEM((_CH, _DEGW), jnp.float32),
        pltpu.VMEM((_ZCH, _DH), jnp.float32),
        pltpu.VMEM((_S_PT, _DEGW), jnp.float32),
        pltpu.VMEM_SHARED((_XE_PAD, _DH), jnp.float32),
        pltpu.VMEM_SHARED((_S_PAD, _DEGW), jnp.float32),
        pltpu.SemaphoreType.DMA,
        pltpu.SemaphoreType.DMA,
    ],
)(_sc1_body)


def _sc2_body(xe0, xe1, vtx3, edg3, zrow,
              s_out,
              vtx_v, edg_v, rowbuf, rowbuf2, zbuf, s_acc, gsem, gsem2):
    c = lax.axis_index("c")
    s = lax.axis_index("s")
    pltpu.sync_copy(zrow, zbuf)
    for k in range(_S_PT // _ZCH):
        pltpu.sync_copy(zbuf, s_acc.at[pl.ds(s * _S_PT + k * _ZCH, _ZCH)])
    plsc.subcore_barrier()

    def run(xe_h):
        def block(b, carry):
            pltpu.sync_copy(vtx3.at[s, pl.ds(b * _NB, _NB)], vtx_v)
            pltpu.sync_copy(edg3.at[s, pl.ds(b * _NB, _NB)], edg_v)

            def pair(p, c2):
                j0 = 2 * p
                j1 = j0 + 1
                cp0 = pltpu.async_copy(xe_h.at[edg_v.at[j0]], rowbuf, gsem)
                cp1 = pltpu.async_copy(xe_h.at[edg_v.at[j1]], rowbuf2, gsem2)
                cp0.wait()
                pltpu.sync_copy(rowbuf, s_acc.at[vtx_v.at[j0]], add=True)
                cp1.wait()
                pltpu.sync_copy(rowbuf2, s_acc.at[vtx_v.at[j1]], add=True)
                return c2
            lax.fori_loop(0, _NB // 2, pair, 0)
            return carry
        lax.fori_loop(0, _NBLK, block, 0)

    @pl.when(c == 0)
    def _():
        run(xe0)

    @pl.when(c == 1)
    def _():
        run(xe1)

    plsc.subcore_barrier()
    for k in range(_S_PT // _ZCH):
        rows = pl.ds(s * _S_PT + k * _ZCH, _ZCH)
        pltpu.sync_copy(s_acc.at[rows], zbuf)
        pltpu.sync_copy(zbuf, s_out.at[rows, pl.ds(c * _DH, _DH)])


_sc_phase2 = functools.partial(
    pl.kernel,
    out_type=jax.ShapeDtypeStruct((_S_PAD, _D), jnp.float32),
    mesh=_MESH,
    compiler_params=_SC_PARAMS,
    scratch_types=[
        pltpu.VMEM((_NB, _CH), jnp.int32),
        pltpu.VMEM((_NB, _CH), jnp.int32),
        pltpu.VMEM((_CH, _DH), jnp.float32),
        pltpu.VMEM((_CH, _DH), jnp.float32),
        pltpu.VMEM((_ZCH, _DH), jnp.float32),
        pltpu.VMEM_SHARED((_S_PAD, _DH), jnp.float32),
        pltpu.SemaphoreType.DMA,
        pltpu.SemaphoreType.DMA,
    ],
)(_sc2_body)


def _final_body(x_ref, x0_ref, s_ref, deg_ref,
                w2a_ref, w2b_ref, b2_ref, ww_ref, wb_ref, o_ref):
    d = deg_ref[...][:, 0:1]
    xv = jnp.dot(x_ref[...] * d, w2a_ref[...], preferred_element_type=jnp.float32)
    xv = xv + jnp.dot(s_ref[...], w2b_ref[...], preferred_element_type=jnp.float32)
    xv = xv + d * b2_ref[...]
    xmix = (1.0 - _ALPHA) * xv + _ALPHA * x0_ref[...]
    o_ref[...] = jnp.dot(xmix, ww_ref[...], preferred_element_type=jnp.float32) + wb_ref[...]


def _final(X, X0, S, deg, W2a, W2b, b2, W_w, W_b):
    blk = 1000
    full = lambda i: (0, 0)
    return pl.pallas_call(
        _final_body,
        grid=(_N_NODES // blk,),
        in_specs=[
            pl.BlockSpec((blk, _D), lambda i: (i, 0)),
            pl.BlockSpec((blk, _D), lambda i: (i, 0)),
            pl.BlockSpec((blk, _D), lambda i: (i, 0)),
            pl.BlockSpec((blk, _DEGW), lambda i: (i, 0)),
            pl.BlockSpec((_D, _D), full),
            pl.BlockSpec((_D, _D), full),
            pl.BlockSpec((1, _D), full),
            pl.BlockSpec((_D, _D), full),
            pl.BlockSpec((1, _D), full),
        ],
        out_specs=pl.BlockSpec((blk, _D), lambda i: (i, 0)),
        out_shape=jax.ShapeDtypeStruct((_N_NODES, _D), jnp.float32),
    )(X, X0, S, deg, W2a, W2b, b2, W_w, W_b)


def kernel(X, vertex, edges, X0, W1_w, W1_b, W2_w, W2_b, W_w, W_b):
    vertex = vertex.astype(jnp.int32)
    edges = edges.astype(jnp.int32)
    vtx3 = vertex.reshape(_NTILES, _NCH, _CH)
    edg3 = edges.reshape(_NTILES, _NCH, _CH)

    xw1a, xw1b = _xw1(X, W1_w, W1_b.reshape(1, _D))

    zrow = jnp.zeros((_ZCH, _DH), jnp.float32)
    zdeg = jnp.zeros((_S_PT, _DEGW), jnp.float32)
    ones_h = jnp.ones((_CH, _DEGW), jnp.float32)

    xe0, xe1, deg = _sc_phase1(xw1a, xw1b, vtx3, edg3, zrow, zdeg, ones_h)
    S = _sc_phase2(xe0, xe1, vtx3, edg3, zrow)
    S = S[:_N_NODES]
    deg = deg[:_N_NODES]

    out = _final(X, X0, S, deg,
                 W2_w[:_D], W2_w[_D:],
                 W2_b.reshape(1, _D), W_w, W_b.reshape(1, _D))
    return out
